# Initial kernel scaffold; baseline (speedup 1.0000x reference)
import jax
import jax.numpy as jnp
from jax.experimental import pallas as pl


def kernel(pred, target):
    raise NotImplementedError("write your pallas kernel here")


# speedup vs baseline: 1.1769x; 1.1769x over previous
import numpy as np
import jax
import jax.numpy as jnp
from jax.experimental import pallas as pl
from jax.experimental.pallas import tpu as pltpu

_BINS = 10
_N = 4194304
_TOT = float(_N)
_ROWS = 4096
_COLS = 1024
_BLOCK_ROWS = 512

_BORDERS = np.arange(_BINS + 1, dtype=np.float32) / _BINS


def _ghm_kernel(p_ref, t_ref, out_ref, acc_ref):
    pi = pl.program_id(0)
    nprog = pl.num_programs(0)

    @pl.when(pi == 0)
    def _init():
        for j in range(2):
            for i in range(_BINS):
                acc_ref[j, i] = jnp.float32(0.0)

    p = p_ref[...]
    t = t_ref[...]
    s = jax.nn.sigmoid(p)
    g = jnp.abs(s - t)
    le = jnp.maximum(p, 0.0) - p * t + jnp.log1p(jnp.exp(-jnp.abs(p)))

    acc_ref[1, 0] += jnp.sum(le)
    for i in range(1, _BINS):
        m = g >= _BORDERS[i]
        acc_ref[0, i] += jnp.sum(m.astype(jnp.float32))
        acc_ref[1, i] += jnp.sum(jnp.where(m, le, 0.0))

    @pl.when(pi == nprog - 1)
    def _finalize():
        dm = jnp.float32(1.0 - 0.9)
        loss_acc = jnp.float32(0.0)
        n_count = jnp.float32(0.0)
        for i in range(_BINS):
            c_lo = jnp.float32(_TOT) if i == 0 else acc_ref[0, i]
            c_hi = jnp.float32(0.0) if i == _BINS - 1 else acc_ref[0, i + 1]
            s_lo = acc_ref[1, i] if i > 0 else acc_ref[1, 0]
            s_hi = jnp.float32(0.0) if i == _BINS - 1 else acc_ref[1, i + 1]
            num = c_lo - c_hi
            sb = s_lo - s_hi
            accm = dm * num
            w = jnp.where(num > 0, _TOT / jnp.maximum(accm, 1e-12), 0.0)
            loss_acc = loss_acc + w * sb
            n_count = n_count + (num > 0).astype(jnp.float32)
        out_ref[0, 0] = loss_acc / jnp.maximum(n_count, 1.0) / _TOT


@jax.jit
def _ghm(pred, target):
    p = pred.reshape(_ROWS, _COLS)
    t = target.astype(jnp.float32).reshape(_ROWS, _COLS)
    grid = (_ROWS // _BLOCK_ROWS,)
    out = pl.pallas_call(
        _ghm_kernel,
        grid=grid,
        in_specs=[
            pl.BlockSpec((_BLOCK_ROWS, _COLS), lambda i: (i, 0)),
            pl.BlockSpec((_BLOCK_ROWS, _COLS), lambda i: (i, 0)),
        ],
        out_specs=pl.BlockSpec(
            (1, 1), lambda i: (0, 0), memory_space=pltpu.SMEM
        ),
        out_shape=jax.ShapeDtypeStruct((1, 1), jnp.float32),
        scratch_shapes=[pltpu.SMEM((2, _BINS), jnp.float32)],
        compiler_params=pltpu.CompilerParams(
            dimension_semantics=("arbitrary",)
        ),
    )(p, t)
    return out[0, 0]


def kernel(pred, target):
    return _ghm(pred, target)


# speedup vs baseline: 1.5271x; 1.2975x over previous
import numpy as np
import jax
import jax.numpy as jnp
from jax.experimental import pallas as pl
from jax.experimental.pallas import tpu as pltpu

_BINS = 10
_N = 4194304
_TOT = float(_N)
_ROWS = 4096
_COLS = 1024
_BLOCK_ROWS = 256
_CHUNK = 8

_BORDERS = np.arange(_BINS + 1, dtype=np.float32) / _BINS


def _fold(x):
    acc = x[:, 0:128]
    for k in range(1, _COLS // 128):
        acc = acc + x[:, k * 128:(k + 1) * 128]
    return acc


def _ghm_kernel(p_ref, t_ref, out_ref, acc_ref):
    pi = pl.program_id(0)
    nprog = pl.num_programs(0)

    zero = jnp.zeros((_CHUNK, 128), jnp.float32)
    s_acc = [zero] * _BINS
    c_acc = [zero] * _BINS

    for j in range(_BLOCK_ROWS // _CHUNK):
        p = p_ref[pl.ds(j * _CHUNK, _CHUNK), :]
        t = t_ref[pl.ds(j * _CHUNK, _CHUNK), :]
        ap = jnp.abs(p)
        e = jnp.exp(-ap)
        r = 1.0 / (1.0 + e)
        s = jnp.where(p >= 0.0, r, e * r)
        g = jnp.abs(s - t)
        le = jnp.maximum(p, 0.0) - p * t - jnp.log(r)
        s_acc[0] = s_acc[0] + _fold(le)
        for i in range(1, _BINS):
            m = g >= _BORDERS[i]
            c_acc[i] = c_acc[i] + _fold(jnp.where(m, 1.0, 0.0))
            s_acc[i] = s_acc[i] + _fold(jnp.where(m, le, 0.0))

    @pl.when(pi == 0)
    def _init():
        for i in range(_BINS):
            acc_ref[i] = s_acc[i]
        for i in range(1, _BINS):
            acc_ref[_BINS + i] = c_acc[i]

    @pl.when(pi > 0)
    def _accum():
        for i in range(_BINS):
            acc_ref[i] += s_acc[i]
        for i in range(1, _BINS):
            acc_ref[_BINS + i] += c_acc[i]

    @pl.when(pi == nprog - 1)
    def _finalize():
        dm = jnp.float32(1.0 - 0.9)
        s_tot = [jnp.sum(acc_ref[i]) for i in range(_BINS)]
        c_tot = [jnp.float32(_TOT)] + [
            jnp.sum(acc_ref[_BINS + i]) for i in range(1, _BINS)
        ]
        loss_acc = jnp.float32(0.0)
        n_count = jnp.float32(0.0)
        for i in range(_BINS):
            c_hi = jnp.float32(0.0) if i == _BINS - 1 else c_tot[i + 1]
            s_hi = jnp.float32(0.0) if i == _BINS - 1 else s_tot[i + 1]
            num = c_tot[i] - c_hi
            sb = s_tot[i] - s_hi
            accm = dm * num
            w = jnp.where(num > 0, _TOT / jnp.maximum(accm, 1e-12), 0.0)
            loss_acc = loss_acc + w * sb
            n_count = n_count + (num > 0).astype(jnp.float32)
        out_ref[0, 0] = loss_acc / jnp.maximum(n_count, 1.0) / _TOT


@jax.jit
def _ghm(pred, target):
    p = pred.reshape(_ROWS, _COLS)
    t = target.astype(jnp.float32).reshape(_ROWS, _COLS)
    grid = (_ROWS // _BLOCK_ROWS,)
    out = pl.pallas_call(
        _ghm_kernel,
        grid=grid,
        in_specs=[
            pl.BlockSpec((_BLOCK_ROWS, _COLS), lambda i: (i, 0)),
            pl.BlockSpec((_BLOCK_ROWS, _COLS), lambda i: (i, 0)),
        ],
        out_specs=pl.BlockSpec(
            (1, 1), lambda i: (0, 0), memory_space=pltpu.SMEM
        ),
        out_shape=jax.ShapeDtypeStruct((1, 1), jnp.float32),
        scratch_shapes=[pltpu.VMEM((2 * _BINS, _CHUNK, 128), jnp.float32)],
        compiler_params=pltpu.CompilerParams(
            dimension_semantics=("arbitrary",)
        ),
    )(p, t)
    return out[0, 0]


def kernel(pred, target):
    return _ghm(pred, target)
